# Initial kernel scaffold; baseline (speedup 1.0000x reference)
import jax
import jax.numpy as jnp
from jax.experimental import pallas as pl


def kernel(x, edge_index, W1, b1, W2, b2):
    raise NotImplementedError("write your pallas kernel here")


# speedup vs baseline: 9.6165x; 9.6165x over previous
import functools

import jax
import jax.numpy as jnp
from jax import lax
from jax.experimental import pallas as pl
from jax.experimental.pallas import tpu as pltpu
from jax.experimental.pallas import tpu_sc as plsc

N = 10000
E = 320000
D = 128
DOUT = 64
NC = 2
NS = 16
NW = NC * NS
EPT = E // NW
C = 125
SW = 16
NSC = EPT // (C * SW)
RPT = N // NS
ZR = 125
DEGW = 16


def _sc_agg_body(with_deg, dw, *refs):
    if with_deg:
        (feat_hbm, src_hbm, dst_hbm, out_hbm, deg_hbm,
         src_v, dst_v, rows_v, onesb, acc_sh, deg_sh, sem) = refs
    else:
        (feat_hbm, src_hbm, dst_hbm, out_hbm,
         src_v, dst_v, rows_v, acc_sh, sem) = refs

    cid = lax.axis_index("c")
    sid = lax.axis_index("s")
    wid = cid * NS + sid
    row0 = sid * RPT

    z16 = jnp.zeros((16,), jnp.float32)

    def zinit(i, carry):
        for j in range(dw // 16):
            rows_v[i, pl.ds(j * 16, 16)] = z16
        if with_deg:
            onesb[i, pl.ds(0, DEGW)] = z16
        return carry

    lax.fori_loop(0, ZR, zinit, 0)
    for r in range(RPT // ZR):
        pltpu.sync_copy(rows_v.at[pl.ds(0, ZR)],
                        acc_sh.at[pl.ds(row0 + r * ZR, ZR)])
        if with_deg:
            pltpu.sync_copy(onesb, deg_sh.at[pl.ds(row0 + r * ZR, ZR)])

    if with_deg:
        o16 = jnp.ones((16,), jnp.float32)

        def oinit(i, carry):
            onesb[i, pl.ds(0, DEGW)] = o16
            return carry

        lax.fori_loop(0, C, oinit, 0)

    plsc.subcore_barrier()

    def super_step(s, carry):
        pltpu.sync_copy(src_hbm.at[wid, pl.ds(s * SW, SW)], src_v)
        pltpu.sync_copy(dst_hbm.at[wid, pl.ds(s * SW, SW)], dst_v)

        def step(k, carry2):
            pltpu.async_copy(feat_hbm.at[src_v.at[k]], rows_v, sem).wait()
            pltpu.sync_copy(rows_v, acc_sh.at[dst_v.at[k]], add=True)
            if with_deg:
                pltpu.sync_copy(onesb, deg_sh.at[dst_v.at[k]], add=True)
            return carry2

        return lax.fori_loop(0, SW, step, carry)

    lax.fori_loop(0, NSC, super_step, 0)
    plsc.subcore_barrier()

    wrows = N // 10

    @pl.when(sid < 10)
    def _write():
        w0 = sid * wrows
        pltpu.sync_copy(acc_sh.at[pl.ds(w0, wrows)],
                        out_hbm.at[cid, pl.ds(w0, wrows)])
        if with_deg:
            pltpu.sync_copy(deg_sh.at[pl.ds(w0, wrows)],
                            deg_hbm.at[cid, pl.ds(w0, wrows)])


def _make_sc_agg(dw, with_deg):
    mesh = plsc.VectorSubcoreMesh(core_axis_name="c", subcore_axis_name="s")
    out_type = [jax.ShapeDtypeStruct((NC, N, dw), jnp.float32)]
    scratch = [
        pltpu.VMEM((SW, C), jnp.int32),
        pltpu.VMEM((SW, C), jnp.int32),
        pltpu.VMEM((C, dw), jnp.float32),
    ]
    if with_deg:
        out_type.append(jax.ShapeDtypeStruct((NC, N, DEGW), jnp.float32))
        scratch.append(pltpu.VMEM((ZR, DEGW), jnp.float32))
    scratch.append(pltpu.VMEM_SHARED((N, dw), jnp.float32))
    if with_deg:
        scratch.append(pltpu.VMEM_SHARED((N, DEGW), jnp.float32))
    scratch.append(pltpu.SemaphoreType.DMA)
    return pl.kernel(
        functools.partial(_sc_agg_body, with_deg, dw),
        out_type=out_type,
        mesh=mesh,
        scratch_types=scratch,
        compiler_params=pltpu.CompilerParams(use_tc_tiling_on_sc=False),
    )


_sc_agg_deg = _make_sc_agg(D, True)
_sc_agg_out = _make_sc_agg(DOUT, False)


def _mm_body(x_ref, w_ref, o_ref):
    o_ref[...] = jnp.dot(x_ref[...], w_ref[...],
                         preferred_element_type=jnp.float32)


def _tc_matmul(x, w, bm=2000):
    m, k = x.shape
    n = w.shape[1]
    return pl.pallas_call(
        _mm_body,
        grid=(m // bm,),
        in_specs=[pl.BlockSpec((bm, k), lambda i: (i, 0)),
                  pl.BlockSpec((k, n), lambda i: (0, 0))],
        out_specs=pl.BlockSpec((bm, n), lambda i: (i, 0)),
        out_shape=jax.ShapeDtypeStruct((m, n), jnp.float32),
    )(x, w)


def _mid_body(sp_ref, degp_ref, b1_ref, w2_ref, o_ref):
    s = sp_ref[0] + sp_ref[1]
    deg = jnp.maximum(degp_ref[0, :, :1] + degp_ref[1, :, :1], 1.0)
    h = jnp.maximum(s / deg + b1_ref[...], 0.0)
    o_ref[...] = jnp.dot(h, w2_ref[...], preferred_element_type=jnp.float32)


def _tc_mid(sp, degp, b1, w2, bm=2000):
    return pl.pallas_call(
        _mid_body,
        grid=(N // bm,),
        in_specs=[pl.BlockSpec((NC, bm, D), lambda i: (0, i, 0)),
                  pl.BlockSpec((NC, bm, DEGW), lambda i: (0, i, 0)),
                  pl.BlockSpec((1, D), lambda i: (0, 0)),
                  pl.BlockSpec((D, DOUT), lambda i: (0, 0))],
        out_specs=pl.BlockSpec((bm, DOUT), lambda i: (i, 0)),
        out_shape=jax.ShapeDtypeStruct((N, DOUT), jnp.float32),
    )(sp, degp, b1, w2)


def _out_body(sp_ref, degp_ref, b2_ref, o_ref):
    s = sp_ref[0] + sp_ref[1]
    deg = jnp.maximum(degp_ref[0, :, :1] + degp_ref[1, :, :1], 1.0)
    o_ref[...] = s / deg + b2_ref[...]


def _tc_out(sp, degp, b2, bm=2000):
    return pl.pallas_call(
        _out_body,
        grid=(N // bm,),
        in_specs=[pl.BlockSpec((NC, bm, DOUT), lambda i: (0, i, 0)),
                  pl.BlockSpec((NC, bm, DEGW), lambda i: (0, i, 0)),
                  pl.BlockSpec((1, DOUT), lambda i: (0, 0))],
        out_specs=pl.BlockSpec((bm, DOUT), lambda i: (i, 0)),
        out_shape=jax.ShapeDtypeStruct((N, DOUT), jnp.float32),
    )(sp, degp, b2)


def kernel(x, edge_index, W1, b1, W2, b2):
    src3 = edge_index[0].reshape(NW, NSC * SW, C)
    dst3 = edge_index[1].reshape(NW, NSC * SW, C)
    xw = _tc_matmul(x, W1)
    s1p, degp = _sc_agg_deg(xw, src3, dst3)
    hw = _tc_mid(s1p, degp, b1.reshape(1, D), W2)
    (s2p,) = _sc_agg_out(hw, src3, dst3)
    return _tc_out(s2p, degp, b2.reshape(1, DOUT))


# speedup vs baseline: 10.8679x; 1.1301x over previous
import functools

import jax
import jax.numpy as jnp
from jax import lax
from jax.experimental import pallas as pl
from jax.experimental.pallas import tpu as pltpu
from jax.experimental.pallas import tpu_sc as plsc

N = 10000
E = 320000
D = 128
DOUT = 64
NC = 2
NS = 16
NW = NC * NS
EPT = E // NW
C = 80
NCH = EPT // C
RPT = N // NS
DEGW = 16
NWR = 10
WR = N // NWR


def _zero_rows(buf, nrows, width):
    z16 = jnp.zeros((16,), jnp.float32)

    def zinit(i, carry):
        for j in range(width // 16):
            buf[i, pl.ds(j * 16, 16)] = z16
        return carry

    lax.fori_loop(0, nrows, zinit, 0)


def _zero_acc_slice(stage, acc_sh, row0):
    for r in range(RPT // C):
        pltpu.sync_copy(stage, acc_sh.at[pl.ds(row0 + r * C, C)])
    rem = RPT - (RPT // C) * C
    pltpu.sync_copy(stage.at[pl.ds(0, rem)],
                    acc_sh.at[pl.ds(row0 + (RPT // C) * C, rem)])


def _sc_agg_body(dw, feat_hbm, src_hbm, dst_hbm, out_hbm,
                 src_v, dst_v, rows_a, rows_b, acc_sh, sem_a, sem_b):
    cid = lax.axis_index("c")
    sid = lax.axis_index("s")
    wid = cid * NS + sid
    row0 = sid * RPT

    _zero_rows(rows_a, C, dw)
    _zero_acc_slice(rows_a, acc_sh, row0)
    pltpu.sync_copy(src_hbm.at[wid], src_v)
    pltpu.sync_copy(dst_hbm.at[wid], dst_v)
    plsc.subcore_barrier()

    pltpu.async_copy(feat_hbm.at[src_v.at[0]], rows_a, sem_a)

    def pair(p, carry):
        c0 = 2 * p
        pltpu.make_async_copy(feat_hbm.at[src_v.at[c0]], rows_a, sem_a).wait()
        pltpu.async_copy(feat_hbm.at[src_v.at[c0 + 1]], rows_b, sem_b)
        pltpu.sync_copy(rows_a, acc_sh.at[dst_v.at[c0]], add=True)
        pltpu.make_async_copy(feat_hbm.at[src_v.at[c0 + 1]], rows_b,
                              sem_b).wait()

        @pl.when(c0 + 2 < NCH)
        def _prefetch():
            pltpu.async_copy(feat_hbm.at[src_v.at[c0 + 2]], rows_a, sem_a)

        pltpu.sync_copy(rows_b, acc_sh.at[dst_v.at[c0 + 1]], add=True)
        return carry

    lax.fori_loop(0, NCH // 2, pair, 0)
    if NCH % 2:
        pltpu.make_async_copy(feat_hbm.at[src_v.at[NCH - 1]], rows_a,
                              sem_a).wait()
        pltpu.sync_copy(rows_a, acc_sh.at[dst_v.at[NCH - 1]], add=True)
    plsc.subcore_barrier()

    @pl.when(sid < NWR)
    def _write():
        w0 = sid * WR
        pltpu.sync_copy(acc_sh.at[pl.ds(w0, WR)],
                        out_hbm.at[cid, pl.ds(w0, WR)])


def _make_sc_agg(dw):
    mesh = plsc.VectorSubcoreMesh(core_axis_name="c", subcore_axis_name="s")
    return pl.kernel(
        functools.partial(_sc_agg_body, dw),
        out_type=[jax.ShapeDtypeStruct((NC, N, dw), jnp.float32)],
        mesh=mesh,
        scratch_types=[
            pltpu.VMEM((NCH, C), jnp.int32),
            pltpu.VMEM((NCH, C), jnp.int32),
            pltpu.VMEM((C, dw), jnp.float32),
            pltpu.VMEM((C, dw), jnp.float32),
            pltpu.VMEM_SHARED((N, dw), jnp.float32),
            pltpu.SemaphoreType.DMA,
            pltpu.SemaphoreType.DMA,
        ],
        compiler_params=pltpu.CompilerParams(use_tc_tiling_on_sc=False),
    )


def _sc_deg_body(dst_hbm, deg_hbm, dst_v, onesb, deg_sh, sem):
    cid = lax.axis_index("c")
    sid = lax.axis_index("s")
    wid = cid * NS + sid
    row0 = sid * RPT

    _zero_rows(onesb, C, DEGW)
    _zero_acc_slice(onesb, deg_sh, row0)
    o16 = jnp.ones((16,), jnp.float32)

    def oinit(i, carry):
        onesb[i, pl.ds(0, DEGW)] = o16
        return carry

    lax.fori_loop(0, C, oinit, 0)
    pltpu.sync_copy(dst_hbm.at[wid], dst_v)
    plsc.subcore_barrier()

    def step(k, carry):
        pltpu.async_copy(onesb, deg_sh.at[dst_v.at[k]], sem, add=True)

        @pl.when(k >= 8)
        def _lagged_wait():
            pltpu.make_async_copy(onesb, deg_sh.at[dst_v.at[0]], sem).wait()

        return carry

    lax.fori_loop(0, NCH, step, 0)
    for _ in range(8):
        pltpu.make_async_copy(onesb, deg_sh.at[dst_v.at[0]], sem).wait()
    plsc.subcore_barrier()

    @pl.when(sid < NWR)
    def _write():
        w0 = sid * WR
        pltpu.sync_copy(deg_sh.at[pl.ds(w0, WR)],
                        deg_hbm.at[cid, pl.ds(w0, WR)])


_sc_deg = pl.kernel(
    _sc_deg_body,
    out_type=[jax.ShapeDtypeStruct((NC, N, DEGW), jnp.float32)],
    mesh=plsc.VectorSubcoreMesh(core_axis_name="c", subcore_axis_name="s"),
    scratch_types=[
        pltpu.VMEM((NCH, C), jnp.int32),
        pltpu.VMEM((C, DEGW), jnp.float32),
        pltpu.VMEM_SHARED((N, DEGW), jnp.float32),
        pltpu.SemaphoreType.DMA,
    ],
    compiler_params=pltpu.CompilerParams(use_tc_tiling_on_sc=False),
)

_sc_agg_d = _make_sc_agg(D)
_sc_agg_o = _make_sc_agg(DOUT)


def _mm_body(x_ref, w_ref, o_ref):
    o_ref[...] = jnp.dot(x_ref[...], w_ref[...],
                         preferred_element_type=jnp.float32)


def _tc_matmul(x, w, bm=2000):
    m, k = x.shape
    n = w.shape[1]
    return pl.pallas_call(
        _mm_body,
        grid=(m // bm,),
        in_specs=[pl.BlockSpec((bm, k), lambda i: (i, 0)),
                  pl.BlockSpec((k, n), lambda i: (0, 0))],
        out_specs=pl.BlockSpec((bm, n), lambda i: (i, 0)),
        out_shape=jax.ShapeDtypeStruct((m, n), jnp.float32),
    )(x, w)


def _mid_body(sp_ref, degp_ref, b1_ref, w2_ref, o_ref):
    s = sp_ref[0] + sp_ref[1]
    deg = jnp.maximum(degp_ref[0, :, :1] + degp_ref[1, :, :1], 1.0)
    h = jnp.maximum(s / deg + b1_ref[...], 0.0)
    o_ref[...] = jnp.dot(h, w2_ref[...], preferred_element_type=jnp.float32)


def _tc_mid(sp, degp, b1, w2, bm=2000):
    return pl.pallas_call(
        _mid_body,
        grid=(N // bm,),
        in_specs=[pl.BlockSpec((NC, bm, D), lambda i: (0, i, 0)),
                  pl.BlockSpec((NC, bm, DEGW), lambda i: (0, i, 0)),
                  pl.BlockSpec((1, D), lambda i: (0, 0)),
                  pl.BlockSpec((D, DOUT), lambda i: (0, 0))],
        out_specs=pl.BlockSpec((bm, DOUT), lambda i: (i, 0)),
        out_shape=jax.ShapeDtypeStruct((N, DOUT), jnp.float32),
    )(sp, degp, b1, w2)


def _out_body(sp_ref, degp_ref, b2_ref, o_ref):
    s = sp_ref[0] + sp_ref[1]
    deg = jnp.maximum(degp_ref[0, :, :1] + degp_ref[1, :, :1], 1.0)
    o_ref[...] = s / deg + b2_ref[...]


def _tc_out(sp, degp, b2, bm=2000):
    return pl.pallas_call(
        _out_body,
        grid=(N // bm,),
        in_specs=[pl.BlockSpec((NC, bm, DOUT), lambda i: (0, i, 0)),
                  pl.BlockSpec((NC, bm, DEGW), lambda i: (0, i, 0)),
                  pl.BlockSpec((1, DOUT), lambda i: (0, 0))],
        out_specs=pl.BlockSpec((bm, DOUT), lambda i: (i, 0)),
        out_shape=jax.ShapeDtypeStruct((N, DOUT), jnp.float32),
    )(sp, degp, b2)


def kernel(x, edge_index, W1, b1, W2, b2):
    src3 = edge_index[0].reshape(NW, NCH, C)
    dst3 = edge_index[1].reshape(NW, NCH, C)
    (degp,) = _sc_deg(dst3)
    xw = _tc_matmul(x, W1)
    (s1p,) = _sc_agg_d(xw, src3, dst3)
    hw = _tc_mid(s1p, degp, b1.reshape(1, D), W2)
    (s2p,) = _sc_agg_o(hw, src3, dst3)
    return _tc_out(s2p, degp, b2.reshape(1, DOUT))


# speedup vs baseline: 12.1104x; 1.1143x over previous
import functools

import jax
import jax.numpy as jnp
from jax import lax
from jax.experimental import pallas as pl
from jax.experimental.pallas import tpu as pltpu
from jax.experimental.pallas import tpu_sc as plsc

N = 10000
E = 320000
D = 128
DOUT = 64
NC = 2
NS = 16
NW = NC * NS
EPT = E // NW
C = 125
SW = 16
NSC = EPT // (C * SW)
NCH = SW * NSC
RPT = N // NS
DEGW = 16
NWR = 10
WR = N // NWR


def _zero_rows(buf, nrows, width):
    z16 = jnp.zeros((16,), jnp.float32)

    def zinit(i, carry):
        for j in range(width // 16):
            buf[i, pl.ds(j * 16, 16)] = z16
        return carry

    lax.fori_loop(0, nrows, zinit, 0)


def _zero_acc_slice(stage, srows, acc_sh, row0):
    nfull = RPT // srows
    for r in range(nfull):
        pltpu.sync_copy(stage, acc_sh.at[pl.ds(row0 + r * srows, srows)])
    rem = RPT - nfull * srows
    if rem:
        pltpu.sync_copy(stage.at[pl.ds(0, rem)],
                        acc_sh.at[pl.ds(row0 + nfull * srows, rem)])


def _sc_agg_body(dw, feat_hbm, src_hbm, dst_hbm, out_hbm,
                 src_v, dst_v, rows_a, rows_b, acc_sh,
                 gsem_a, gsem_b, ssem_a, ssem_b):
    cid = lax.axis_index("c")
    sid = lax.axis_index("s")
    wid = cid * NS + sid
    row0 = sid * RPT

    _zero_rows(rows_a, C, dw)
    _zero_acc_slice(rows_a, C, acc_sh, row0)
    plsc.subcore_barrier()

    def gather(j, buf, sem):
        pltpu.async_copy(feat_hbm.at[src_v.at[j]], buf, sem)

    def gwait(j, buf, sem):
        pltpu.make_async_copy(feat_hbm.at[src_v.at[j]], buf, sem).wait()

    def scat(j, buf, sem):
        pltpu.async_copy(buf, acc_sh.at[dst_v.at[j]], sem, add=True)

    def swait(buf, sem):
        pltpu.make_async_copy(buf, acc_sh.at[dst_v.at[0]], sem).wait()

    def super_step(s, carry):
        @pl.when(s > 0)
        def _drain_prev():
            swait(rows_a, ssem_a)
            swait(rows_b, ssem_b)

        pltpu.sync_copy(src_hbm.at[wid, pl.ds(s * SW, SW)], src_v)
        pltpu.sync_copy(dst_hbm.at[wid, pl.ds(s * SW, SW)], dst_v)
        gather(0, rows_a, gsem_a)

        def pair(t, carry2):
            j0 = 2 * t
            gwait(j0, rows_a, gsem_a)

            @pl.when(t > 0)
            def _drain_b():
                swait(rows_b, ssem_b)

            gather(j0 + 1, rows_b, gsem_b)
            scat(j0, rows_a, ssem_a)
            gwait(j0 + 1, rows_b, gsem_b)

            @pl.when(j0 + 2 < SW)
            def _next_a():
                swait(rows_a, ssem_a)
                gather(j0 + 2, rows_a, gsem_a)

            scat(j0 + 1, rows_b, ssem_b)
            return carry2

        lax.fori_loop(0, SW // 2, pair, 0)
        return carry

    lax.fori_loop(0, NSC, super_step, 0)
    swait(rows_a, ssem_a)
    swait(rows_b, ssem_b)
    plsc.subcore_barrier()

    @pl.when(sid < NWR)
    def _write():
        w0 = sid * WR
        pltpu.sync_copy(acc_sh.at[pl.ds(w0, WR)],
                        out_hbm.at[cid, pl.ds(w0, WR)])


def _make_sc_agg(dw):
    mesh = plsc.VectorSubcoreMesh(core_axis_name="c", subcore_axis_name="s")
    return pl.kernel(
        functools.partial(_sc_agg_body, dw),
        out_type=[jax.ShapeDtypeStruct((NC, N, dw), jnp.float32)],
        mesh=mesh,
        scratch_types=[
            pltpu.VMEM((SW, C), jnp.int32),
            pltpu.VMEM((SW, C), jnp.int32),
            pltpu.VMEM((C, dw), jnp.float32),
            pltpu.VMEM((C, dw), jnp.float32),
            pltpu.VMEM_SHARED((N, dw), jnp.float32),
            pltpu.SemaphoreType.DMA,
            pltpu.SemaphoreType.DMA,
            pltpu.SemaphoreType.DMA,
            pltpu.SemaphoreType.DMA,
        ],
        compiler_params=pltpu.CompilerParams(use_tc_tiling_on_sc=False),
    )


def _sc_deg_body(dst_hbm, deg_hbm, dst_v, onesb, deg_sh, sem):
    cid = lax.axis_index("c")
    sid = lax.axis_index("s")
    wid = cid * NS + sid
    row0 = sid * RPT

    _zero_rows(onesb, C, DEGW)
    _zero_acc_slice(onesb, C, deg_sh, row0)
    o16 = jnp.ones((16,), jnp.float32)

    def oinit(i, carry):
        onesb[i, pl.ds(0, DEGW)] = o16
        return carry

    lax.fori_loop(0, C, oinit, 0)
    pltpu.sync_copy(dst_hbm.at[wid], dst_v)
    plsc.subcore_barrier()

    def step(k, carry):
        pltpu.async_copy(onesb, deg_sh.at[dst_v.at[k]], sem, add=True)

        @pl.when(k >= 8)
        def _lagged_wait():
            pltpu.make_async_copy(onesb, deg_sh.at[dst_v.at[0]], sem).wait()

        return carry

    lax.fori_loop(0, NCH, step, 0)
    for _ in range(8):
        pltpu.make_async_copy(onesb, deg_sh.at[dst_v.at[0]], sem).wait()
    plsc.subcore_barrier()

    @pl.when(sid < NWR)
    def _write():
        w0 = sid * WR
        pltpu.sync_copy(deg_sh.at[pl.ds(w0, WR)],
                        deg_hbm.at[cid, pl.ds(w0, WR)])


_sc_deg = pl.kernel(
    _sc_deg_body,
    out_type=[jax.ShapeDtypeStruct((NC, N, DEGW), jnp.float32)],
    mesh=plsc.VectorSubcoreMesh(core_axis_name="c", subcore_axis_name="s"),
    scratch_types=[
        pltpu.VMEM((NCH, C), jnp.int32),
        pltpu.VMEM((C, DEGW), jnp.float32),
        pltpu.VMEM_SHARED((N, DEGW), jnp.float32),
        pltpu.SemaphoreType.DMA,
    ],
    compiler_params=pltpu.CompilerParams(use_tc_tiling_on_sc=False),
)

_sc_agg_d = _make_sc_agg(D)
_sc_agg_o = _make_sc_agg(DOUT)


def _mm_body(x_ref, w_ref, o_ref):
    o_ref[...] = jnp.dot(x_ref[...], w_ref[...],
                         preferred_element_type=jnp.float32)


def _tc_matmul(x, w, bm=2000):
    m, k = x.shape
    n = w.shape[1]
    return pl.pallas_call(
        _mm_body,
        grid=(m // bm,),
        in_specs=[pl.BlockSpec((bm, k), lambda i: (i, 0)),
                  pl.BlockSpec((k, n), lambda i: (0, 0))],
        out_specs=pl.BlockSpec((bm, n), lambda i: (i, 0)),
        out_shape=jax.ShapeDtypeStruct((m, n), jnp.float32),
    )(x, w)


def _mid_body(sp_ref, degp_ref, b1_ref, w2_ref, o_ref):
    s = sp_ref[0] + sp_ref[1]
    deg = jnp.maximum(degp_ref[0, :, :1] + degp_ref[1, :, :1], 1.0)
    h = jnp.maximum(s / deg + b1_ref[...], 0.0)
    o_ref[...] = jnp.dot(h, w2_ref[...], preferred_element_type=jnp.float32)


def _tc_mid(sp, degp, b1, w2, bm=2000):
    return pl.pallas_call(
        _mid_body,
        grid=(N // bm,),
        in_specs=[pl.BlockSpec((NC, bm, D), lambda i: (0, i, 0)),
                  pl.BlockSpec((NC, bm, DEGW), lambda i: (0, i, 0)),
                  pl.BlockSpec((1, D), lambda i: (0, 0)),
                  pl.BlockSpec((D, DOUT), lambda i: (0, 0))],
        out_specs=pl.BlockSpec((bm, DOUT), lambda i: (i, 0)),
        out_shape=jax.ShapeDtypeStruct((N, DOUT), jnp.float32),
    )(sp, degp, b1, w2)


def _out_body(sp_ref, degp_ref, b2_ref, o_ref):
    s = sp_ref[0] + sp_ref[1]
    deg = jnp.maximum(degp_ref[0, :, :1] + degp_ref[1, :, :1], 1.0)
    o_ref[...] = s / deg + b2_ref[...]


def _tc_out(sp, degp, b2, bm=2000):
    return pl.pallas_call(
        _out_body,
        grid=(N // bm,),
        in_specs=[pl.BlockSpec((NC, bm, DOUT), lambda i: (0, i, 0)),
                  pl.BlockSpec((NC, bm, DEGW), lambda i: (0, i, 0)),
                  pl.BlockSpec((1, DOUT), lambda i: (0, 0))],
        out_specs=pl.BlockSpec((bm, DOUT), lambda i: (i, 0)),
        out_shape=jax.ShapeDtypeStruct((N, DOUT), jnp.float32),
    )(sp, degp, b2)


def kernel(x, edge_index, W1, b1, W2, b2):
    src3 = edge_index[0].reshape(NW, NCH, C)
    dst3 = edge_index[1].reshape(NW, NCH, C)
    (degp,) = _sc_deg(dst3)
    xw = _tc_matmul(x, W1)
    (s1p,) = _sc_agg_d(xw, src3, dst3)
    hw = _tc_mid(s1p, degp, b1.reshape(1, D), W2)
    (s2p,) = _sc_agg_o(hw, src3, dst3)
    return _tc_out(s2p, degp, b2.reshape(1, DOUT))
